# baseline (speedup 1.0000x reference)
import functools

import jax
import jax.numpy as jnp
from jax import lax
from jax.experimental import pallas as pl
from jax.experimental.pallas import tpu as pltpu
from jax.experimental.pallas import tpu_sc as plsc

N_U = 2500
N_I = 7500
N = N_U + N_I
D = 128
E_A = 320000
E_S = 80000
K = 96
NC = 2
NS = 16
NW = NC * NS
CA_W = 108
CS_W = 28
WIN = 4
NWIN_A = CA_W // WIN
NWIN_S = CS_W // WIN
RA0, RA_LAST = 624, N - 15 * 624
RS0, RS_LAST = 152, N_U - 15 * 152

_mesh = plsc.VectorSubcoreMesh(core_axis_name="c", subcore_axis_name="s")


@functools.partial(
    pl.kernel,
    out_type=(
        jax.ShapeDtypeStruct((NC, N, D), jnp.float32),
        jax.ShapeDtypeStruct((NC, N_U, D), jnp.float32),
    ),
    mesh=_mesh,
    scratch_types=[
        pltpu.VMEM_SHARED((N, D), jnp.float32),
        pltpu.VMEM_SHARED((N_U, D), jnp.float32),
        [pltpu.VMEM((WIN, K), jnp.int32)] * 2,
        [pltpu.VMEM((WIN, K), jnp.int32)] * 2,
        [pltpu.VMEM((WIN, K), jnp.float32)] * 2,
        [pltpu.VMEM((K, D), jnp.float32)] * 2,
        [pltpu.SemaphoreType.DMA] * 2,
        [pltpu.SemaphoreType.DMA] * 2,
    ],
)
def _sc_spmm(x_hbm, sx_hbm, srca_hbm, dsta_hbm, va_hbm, srcs_hbm, dsts_hbm,
             vs_hbm, pa_hbm, ps_hbm, acc_a, acc_s, wsrc, wdst, wval, rg,
             semg, semw):
    cid = lax.axis_index("c")
    sid = lax.axis_index("s")
    wid = sid * NC + cid

    zero = jnp.zeros((16,), jnp.float32)

    def _zrow(k, _):
        for j in range(D // 16):
            rg[0][k, pl.ds(j * 16, 16)] = zero
        return 0

    lax.fori_loop(0, K, _zrow, 0)

    def _fill(dst, base, n):
        full, rem = n // K, n % K
        for r in range(full):
            pltpu.sync_copy(rg[0].at[pl.ds(0, K)],
                            dst.at[pl.ds(base + r * K, K)])
        if rem:
            pltpu.sync_copy(rg[0].at[pl.ds(0, rem)],
                            dst.at[pl.ds(base + full * K, rem)])

    @pl.when(sid < 15)
    def _():
        _fill(acc_a, sid * RA0, RA0)
        _fill(acc_s, sid * RS0, RS0)

    @pl.when(sid == 15)
    def _():
        _fill(acc_a, 15 * RA0, RA_LAST)
        _fill(acc_s, 15 * RS0, RS_LAST)

    plsc.subcore_barrier()

    def _wait(sem, dummy_src, dref):
        pltpu.make_async_copy(dummy_src, dref, sem).wait()

    def _run_phase(nwin, srch, dsth, valh, x_ref, acc):
        n_chunks = nwin * WIN

        def _load_window(w, p):
            pltpu.async_copy(srch.at[wid, w], wsrc[p], semw[p])
            pltpu.async_copy(dsth.at[wid, w], wdst[p], semw[p])
            pltpu.async_copy(valh.at[wid, w], wval[p], semw[p])

        def _wait_window(p):
            _wait(semw[p], srch.at[0, 0], wsrc[p])
            _wait(semw[p], dsth.at[0, 0], wdst[p])
            _wait(semw[p], valh.at[0, 0], wval[p])

        _load_window(0, 0)
        _wait_window(0)
        _load_window(1, 1)
        pltpu.async_copy(x_ref.at[wsrc[0].at[0]], rg[0], semg[0])
        pltpu.async_copy(x_ref.at[wsrc[0].at[1]], rg[1], semg[1])

        def _window(w, p):
            for c in range(WIN):
                b = c % 2
                i = w * WIN + c
                _wait(semg[b], x_ref.at[pl.ds(0, K)], rg[b])

                def body(g, _):
                    vv = wval[p][c, pl.ds(g * 16, 16)]
                    for e in range(16):
                        v = vv[e]
                        r = g * 16 + e
                        for j in range(D // 16):
                            rg[b][r, pl.ds(j * 16, 16)] = (
                                rg[b][r, pl.ds(j * 16, 16)] * v)
                    return 0

                lax.fori_loop(0, K // 16, body, 0)
                pltpu.sync_copy(rg[b], acc.at[wdst[p].at[c]], add=True)

                if c == WIN - 2:
                    @pl.when(w < nwin - 1)
                    def _():
                        _wait_window(1 - p)

                @pl.when(i + 2 < n_chunks)
                def _():
                    if c < WIN - 2:
                        nsrc = wsrc[p].at[c + 2]
                    else:
                        nsrc = wsrc[1 - p].at[c - (WIN - 2)]
                    pltpu.async_copy(x_ref.at[nsrc], rg[b], semg[b])

                if c == WIN - 1:
                    @pl.when(w + 2 < nwin)
                    def _():
                        _load_window(w + 2, p)

        def _pair(g, _):
            _window(2 * g, 0)
            _window(2 * g + 1, 1)
            return 0

        lax.fori_loop(0, nwin // 2, _pair, 0)
        if nwin % 2:
            _window(nwin - 1, (nwin - 1) % 2)

    _run_phase(NWIN_A, srca_hbm, dsta_hbm, va_hbm, x_hbm, acc_a)
    _run_phase(NWIN_S, srcs_hbm, dsts_hbm, vs_hbm, sx_hbm, acc_s)

    plsc.subcore_barrier()

    @pl.when(sid < 15)
    def _():
        pltpu.sync_copy(acc_a.at[pl.ds(sid * RA0, RA0)],
                        pa_hbm.at[cid, pl.ds(sid * RA0, RA0)])
        pltpu.sync_copy(acc_s.at[pl.ds(sid * RS0, RS0)],
                        ps_hbm.at[cid, pl.ds(sid * RS0, RS0)])

    @pl.when(sid == 15)
    def _():
        pltpu.sync_copy(acc_a.at[pl.ds(15 * RA0, RA_LAST)],
                        pa_hbm.at[cid, pl.ds(15 * RA0, RA_LAST)])
        pltpu.sync_copy(acc_s.at[pl.ds(15 * RS0, RS_LAST)],
                        ps_hbm.at[cid, pl.ds(15 * RS0, RS_LAST)])


def _mish(x):
    sp = jnp.maximum(x, 0.0) + jnp.log(1.0 + jnp.exp(-jnp.abs(x)))
    return x * jnp.tanh(sp)


def _fusion(u, s, f1w, f1b, f2w, f2b, f3w, f3b):
    c = jnp.concatenate([u, s, u * s], axis=1)
    t1 = _mish(jnp.dot(c, f1w, preferred_element_type=jnp.float32) + f1b)
    t2 = _mish(jnp.dot(t1, f2w, preferred_element_type=jnp.float32) + f2b)
    t3 = jnp.dot(t2, f3w, preferred_element_type=jnp.float32) + f3b
    return t3 / jnp.sqrt(jnp.sum(t3 * t3))


def _tc1_body(pa, ps, f1w, f1b, f2w, f2b, f3w, f3b, ego_out):
    a = pa[0] + pa[1]
    s = ps[0] + ps[1]
    u = a[:N_U]
    ego_out[pl.ds(0, N_U), :] = _fusion(u, s, f1w[...], f1b[...], f2w[...],
                                        f2b[...], f3w[...], f3b[...])
    ego_out[pl.ds(N_U, N_I), :] = a[N_U:]


def _tc2_body(qa, qs, ue, ie, ego1, f1w, f1b, f2w, f2b, f3w, f3b,
              user_out, item_out):
    a = qa[0] + qa[1]
    s = qs[0] + qs[1]
    t3n = _fusion(a[:N_U], s, f1w[...], f1b[...], f2w[...], f2b[...],
                  f3w[...], f3b[...])
    user_out[...] = (ue[...] + ego1[pl.ds(0, N_U), :] + t3n) * (1.0 / 3.0)
    item_out[...] = (ie[...] + ego1[pl.ds(N_U, N_I), :] + a[N_U:]) * (1.0 / 3.0)


_tc1 = pl.pallas_call(
    _tc1_body,
    out_shape=jax.ShapeDtypeStruct((N, D), jnp.float32),
)

_tc2 = pl.pallas_call(
    _tc2_body,
    out_shape=(
        jax.ShapeDtypeStruct((N_U, D), jnp.float32),
        jax.ShapeDtypeStruct((N_I, D), jnp.float32),
    ),
)


def _pad_edges(edge_index, vals, chunks_w):
    e = edge_index.shape[1]
    per_w = e // NW
    pad = chunks_w * K - per_w
    src = edge_index[0].reshape(NW, per_w)
    dst = edge_index[1].reshape(NW, per_w)
    v = vals.reshape(NW, per_w)
    zi = jnp.zeros((NW, pad), jnp.int32)
    zf = jnp.zeros((NW, pad), jnp.float32)
    nwin = chunks_w // WIN
    src = jnp.concatenate([src, zi], axis=1).reshape(NW, nwin, WIN, K)
    dst = jnp.concatenate([dst, zi], axis=1).reshape(NW, nwin, WIN, K)
    v = jnp.concatenate([v, zf], axis=1).reshape(NW, nwin, WIN, K)
    return src, dst, v


def kernel(user_emb, item_emb, a_vals, s_vals, fc1_w, fc1_b, fc2_w, fc2_b,
           fc3_w, fc3_b, edge_index_a, edge_index_s):
    x0 = jnp.concatenate([user_emb, item_emb], axis=0)
    src_a, dst_a, va = _pad_edges(edge_index_a, a_vals, CA_W)
    src_s, dst_s, vs = _pad_edges(edge_index_s, s_vals, CS_W)

    pa, ps = _sc_spmm(x0, user_emb, src_a, dst_a, va, src_s, dst_s, vs)
    ego1 = _tc1(pa, ps, fc1_w, fc1_b, fc2_w, fc2_b, fc3_w, fc3_b)
    qa, qs = _sc_spmm(ego1, ego1[:N_U], src_a, dst_a, va, src_s, dst_s, vs)
    user_out, item_out = _tc2(qa, qs, user_emb, item_emb, ego1, fc1_w, fc1_b,
                              fc2_w, fc2_b, fc3_w, fc3_b)
    return user_out, item_out


# speedup vs baseline: 1.0003x; 1.0003x over previous
import functools

import jax
import jax.numpy as jnp
from jax import lax
from jax.experimental import pallas as pl
from jax.experimental.pallas import tpu as pltpu
from jax.experimental.pallas import tpu_sc as plsc

N_U = 2500
N_I = 7500
N = N_U + N_I
D = 128
E_A = 320000
E_S = 80000
K = 96
NC = 2
NS = 16
NW = NC * NS
CA_W = 108
CS_W = 28
WIN = 4
NWIN_A = CA_W // WIN
NWIN_S = CS_W // WIN
RA0, RA_LAST = 624, N - 15 * 624
RS0, RS_LAST = 152, N_U - 15 * 152

_mesh = plsc.VectorSubcoreMesh(core_axis_name="c", subcore_axis_name="s")


@functools.partial(
    pl.kernel,
    out_type=(
        jax.ShapeDtypeStruct((NC, N, D), jnp.float32),
        jax.ShapeDtypeStruct((NC, N_U, D), jnp.float32),
    ),
    mesh=_mesh,
    scratch_types=[
        pltpu.VMEM_SHARED((N, D), jnp.float32),
        pltpu.VMEM_SHARED((N_U, D), jnp.float32),
        [pltpu.VMEM((WIN, K), jnp.int32)] * 2,
        [pltpu.VMEM((WIN, K), jnp.int32)] * 2,
        [pltpu.VMEM((WIN, K), jnp.float32)] * 2,
        [pltpu.VMEM((K, D), jnp.float32)] * 2,
        [pltpu.SemaphoreType.DMA] * 2,
        [pltpu.SemaphoreType.DMA] * 2,
    ],
)
def _sc_spmm(x_hbm, sx_hbm, srca_hbm, dsta_hbm, va_hbm, srcs_hbm, dsts_hbm,
             vs_hbm, pa_hbm, ps_hbm, acc_a, acc_s, wsrc, wdst, wval, rg,
             semg, semw):
    cid = lax.axis_index("c")
    sid = lax.axis_index("s")
    wid = sid * NC + cid

    zero = jnp.zeros((16,), jnp.float32)

    def _zrow(k, _):
        for j in range(D // 16):
            rg[0][k, pl.ds(j * 16, 16)] = zero
        return 0

    lax.fori_loop(0, K, _zrow, 0)

    def _fill(dst, base, n):
        full, rem = n // K, n % K
        for r in range(full):
            pltpu.sync_copy(rg[0].at[pl.ds(0, K)],
                            dst.at[pl.ds(base + r * K, K)])
        if rem:
            pltpu.sync_copy(rg[0].at[pl.ds(0, rem)],
                            dst.at[pl.ds(base + full * K, rem)])

    @pl.when(sid < 15)
    def _():
        _fill(acc_a, sid * RA0, RA0)
        _fill(acc_s, sid * RS0, RS0)

    @pl.when(sid == 15)
    def _():
        _fill(acc_a, 15 * RA0, RA_LAST)
        _fill(acc_s, 15 * RS0, RS_LAST)

    plsc.subcore_barrier()

    def _wait(sem, dummy_src, dref):
        pltpu.make_async_copy(dummy_src, dref, sem).wait()

    def _run_phase(nwin, srch, dsth, valh, x_ref, acc):
        n_chunks = nwin * WIN

        def _load_window(w, p):
            pltpu.async_copy(srch.at[wid, w], wsrc[p], semw[p])
            pltpu.async_copy(dsth.at[wid, w], wdst[p], semw[p])
            pltpu.async_copy(valh.at[wid, w], wval[p], semw[p])

        def _wait_window(p):
            _wait(semw[p], srch.at[0, 0], wsrc[p])
            _wait(semw[p], dsth.at[0, 0], wdst[p])
            _wait(semw[p], valh.at[0, 0], wval[p])

        _load_window(0, 0)
        _wait_window(0)
        _load_window(1, 1)
        pltpu.async_copy(x_ref.at[wsrc[0].at[0]], rg[0], semg[0])
        pltpu.async_copy(x_ref.at[wsrc[0].at[1]], rg[1], semg[1])

        def _window(w, p):
            for c in range(WIN):
                b = c % 2
                i = w * WIN + c
                _wait(semg[b], x_ref.at[pl.ds(0, K)], rg[b])

                def body(g, _):
                    vv = wval[p][c, pl.ds(g * 16, 16)]
                    for e in range(16):
                        v = vv[e]
                        r = g * 16 + e
                        for j in range(D // 16):
                            rg[b][r, pl.ds(j * 16, 16)] = (
                                rg[b][r, pl.ds(j * 16, 16)] * v)
                    return 0

                lax.fori_loop(0, K // 16, body, 0)
                pltpu.sync_copy(rg[b], acc.at[wdst[p].at[c]], add=True)

                if c == WIN - 2:
                    @pl.when(w < nwin - 1)
                    def _():
                        _wait_window(1 - p)

                @pl.when(i + 2 < n_chunks)
                def _():
                    if c < WIN - 2:
                        nsrc = wsrc[p].at[c + 2]
                    else:
                        nsrc = wsrc[1 - p].at[c - (WIN - 2)]
                    pltpu.async_copy(x_ref.at[nsrc], rg[b], semg[b])

                if c == WIN - 1:
                    @pl.when(w + 2 < nwin)
                    def _():
                        _load_window(w + 2, p)

        def _pair(g, _):
            _window(2 * g, 0)
            _window(2 * g + 1, 1)
            return 0

        lax.fori_loop(0, nwin // 2, _pair, 0)
        if nwin % 2:
            _window(nwin - 1, (nwin - 1) % 2)

    _run_phase(NWIN_A, srca_hbm, dsta_hbm, va_hbm, x_hbm, acc_a)
    _run_phase(NWIN_S, srcs_hbm, dsts_hbm, vs_hbm, sx_hbm, acc_s)

    plsc.subcore_barrier()

    @pl.when(sid < 15)
    def _():
        pltpu.sync_copy(acc_a.at[pl.ds(sid * RA0, RA0)],
                        pa_hbm.at[cid, pl.ds(sid * RA0, RA0)])
        pltpu.sync_copy(acc_s.at[pl.ds(sid * RS0, RS0)],
                        ps_hbm.at[cid, pl.ds(sid * RS0, RS0)])

    @pl.when(sid == 15)
    def _():
        pltpu.sync_copy(acc_a.at[pl.ds(15 * RA0, RA_LAST)],
                        pa_hbm.at[cid, pl.ds(15 * RA0, RA_LAST)])
        pltpu.sync_copy(acc_s.at[pl.ds(15 * RS0, RS_LAST)],
                        ps_hbm.at[cid, pl.ds(15 * RS0, RS_LAST)])


def _mish(x):
    sp = jnp.maximum(x, 0.0) + jnp.log(1.0 + jnp.exp(-jnp.abs(x)))
    return x * jnp.tanh(sp)


def _fusion(u, s, f1w, f1b, f2w, f2b, f3w, f3b):
    c = jnp.concatenate([u, s, u * s], axis=1)
    t1 = _mish(jnp.dot(c, f1w, preferred_element_type=jnp.float32) + f1b)
    t2 = _mish(jnp.dot(t1, f2w, preferred_element_type=jnp.float32) + f2b)
    t3 = jnp.dot(t2, f3w, preferred_element_type=jnp.float32) + f3b
    return t3 / jnp.sqrt(jnp.sum(t3 * t3))


def _tc1_body(pa, ps, f1w, f1b, f2w, f2b, f3w, f3b, ego_out):
    a = pa[0] + pa[1]
    s = ps[0] + ps[1]
    u = a[:N_U]
    ego_out[pl.ds(0, N_U), :] = _fusion(u, s, f1w[...], f1b[...], f2w[...],
                                        f2b[...], f3w[...], f3b[...])
    ego_out[pl.ds(N_U, N_I), :] = a[N_U:]


def _tc2_body(qa, qs, ue, ie, ego1, f1w, f1b, f2w, f2b, f3w, f3b,
              user_out, item_out):
    a = qa[0] + qa[1]
    s = qs[0] + qs[1]
    t3n = _fusion(a[:N_U], s, f1w[...], f1b[...], f2w[...], f2b[...],
                  f3w[...], f3b[...])
    user_out[...] = (ue[...] + ego1[pl.ds(0, N_U), :] + t3n) * (1.0 / 3.0)
    item_out[...] = (ie[...] + ego1[pl.ds(N_U, N_I), :] + a[N_U:]) * (1.0 / 3.0)


_tc1 = pl.pallas_call(
    _tc1_body,
    out_shape=jax.ShapeDtypeStruct((N, D), jnp.float32),
)

_tc2 = pl.pallas_call(
    _tc2_body,
    out_shape=(
        jax.ShapeDtypeStruct((N_U, D), jnp.float32),
        jax.ShapeDtypeStruct((N_I, D), jnp.float32),
    ),
)


def _pad_edges(edge_index, vals, chunks_w):
    e = edge_index.shape[1]
    per_w = e // NW
    pad = chunks_w * K - per_w
    src = edge_index[0].reshape(NW, per_w)
    dst = edge_index[1].reshape(NW, per_w)
    v = vals.reshape(NW, per_w)
    zi = jnp.zeros((NW, pad), jnp.int32)
    di = jnp.broadcast_to(jnp.arange(pad, dtype=jnp.int32)[None, :] % N_U,
                          (NW, pad))
    zf = jnp.zeros((NW, pad), jnp.float32)
    nwin = chunks_w // WIN
    src = jnp.concatenate([src, zi], axis=1).reshape(NW, nwin, WIN, K)
    dst = jnp.concatenate([dst, di], axis=1).reshape(NW, nwin, WIN, K)
    v = jnp.concatenate([v, zf], axis=1).reshape(NW, nwin, WIN, K)
    return src, dst, v


def kernel(user_emb, item_emb, a_vals, s_vals, fc1_w, fc1_b, fc2_w, fc2_b,
           fc3_w, fc3_b, edge_index_a, edge_index_s):
    x0 = jnp.concatenate([user_emb, item_emb], axis=0)
    src_a, dst_a, va = _pad_edges(edge_index_a, a_vals, CA_W)
    src_s, dst_s, vs = _pad_edges(edge_index_s, s_vals, CS_W)

    pa, ps = _sc_spmm(x0, user_emb, src_a, dst_a, va, src_s, dst_s, vs)
    ego1 = _tc1(pa, ps, fc1_w, fc1_b, fc2_w, fc2_b, fc3_w, fc3_b)
    qa, qs = _sc_spmm(ego1, ego1[:N_U], src_a, dst_a, va, src_s, dst_s, vs)
    user_out, item_out = _tc2(qa, qs, user_emb, item_emb, ego1, fc1_w, fc1_b,
                              fc2_w, fc2_b, fc3_w, fc3_b)
    return user_out, item_out
